# baseline (speedup 1.0000x reference)
import functools

import jax
import jax.numpy as jnp
from jax import lax
from jax.experimental import pallas as pl
from jax.experimental.pallas import tpu as pltpu
from jax.experimental.pallas import tpu_sc as plsc

N = 10000
NP = 10112
E = 320000
F = 128
NC = 2
NS = 16
NW = NC * NS
EB = 128
NBATCH = 80
E_PAD = NW * NBATCH * EB
SB = 8
NSC = NBATCH // SB
RPT = NP // NS

_mesh = plsc.VectorSubcoreMesh(core_axis_name="c", subcore_axis_name="s")


@functools.partial(
    pl.kernel,
    out_type=jax.ShapeDtypeStruct((NC, NP, F), jnp.float32),
    mesh=_mesh,
    scratch_types=[
        pltpu.VMEM((NBATCH, EB), jnp.int32),
        pltpu.VMEM((NBATCH, EB), jnp.int32),
        pltpu.VMEM((EB, F), jnp.float32),
        pltpu.VMEM_SHARED((NP, F), jnp.float32),
        pltpu.SemaphoreType.DMA,
    ],
)
def _sc_scatter(g_hbm, src_hbm, dst_hbm, z_hbm, out_hbm,
                src_i, dst_i, rows_v, acc, gsem):
    c = lax.axis_index("c")
    s = lax.axis_index("s")
    wid = s * NC + c

    pltpu.sync_copy(z_hbm, acc.at[pl.ds(s * RPT, RPT)])
    pltpu.sync_copy(src_hbm.at[wid], src_i)
    pltpu.sync_copy(dst_hbm.at[wid], dst_i)
    plsc.subcore_barrier()

    for i in range(NBATCH):
        pltpu.async_copy(g_hbm.at[src_i.at[i]], rows_v, gsem).wait()
        pltpu.sync_copy(rows_v, acc.at[dst_i.at[i]], add=True)
    plsc.subcore_barrier()

    pltpu.sync_copy(acc.at[pl.ds(s * RPT, RPT)],
                    out_hbm.at[c, pl.ds(s * RPT, RPT)])


_RB = 2000
_GRID = N // _RB


def _row_spec(cols):
    return pl.BlockSpec((_RB, cols), lambda i: (i, 0))


def _full_spec(shape):
    nd = len(shape)
    return pl.BlockSpec(shape, lambda i: (0,) * nd)


def _entry_body(xg_ref, win_ref, bin_ref, wf_ref, degp_ref, g_ref, dinv_ref):
    dp = degp_ref[0, :, 0:1] + degp_ref[1, :, 0:1] + 1.0
    dinv = lax.rsqrt(jnp.maximum(dp, 1.0))
    h = jnp.dot(xg_ref[...], win_ref[...],
                preferred_element_type=jnp.float32) + bin_ref[...]
    g_ref[...] = dinv * jnp.dot(h, wf_ref[...],
                                preferred_element_type=jnp.float32)
    dinv_ref[...] = jnp.broadcast_to(dinv, (_RB, F))


def _tc_entry(xg, W_in, b_in, W_first, degp):
    return pl.pallas_call(
        _entry_body,
        grid=(_GRID,),
        in_specs=[
            _row_spec(12),
            _full_spec((12, F)),
            _full_spec((F,)),
            _full_spec((F, F)),
            pl.BlockSpec((NC, _RB, F), lambda i: (0, i, 0)),
        ],
        out_specs=[_row_spec(F), _row_spec(F)],
        out_shape=[jax.ShapeDtypeStruct((N, F), jnp.float32),
                   jax.ShapeDtypeStruct((N, F), jnp.float32)],
    )(xg, W_in, b_in, W_first, degp)


def _fused_body(S_ref, g_ref, dinv_ref, b_ref, W_ref, out_ref):
    ssum = S_ref[0] + S_ref[1] + g_ref[...]
    h = jnp.maximum(dinv_ref[...] * ssum + b_ref[...], 0.0)
    out_ref[...] = dinv_ref[...] * jnp.dot(h, W_ref[...],
                                           preferred_element_type=jnp.float32)


def _tc_fused(S, g, dinv, b, W_next):
    return pl.pallas_call(
        _fused_body,
        grid=(_GRID,),
        in_specs=[
            pl.BlockSpec((NC, _RB, F), lambda i: (0, i, 0)),
            _row_spec(F),
            _row_spec(F),
            _full_spec((F,)),
            _full_spec((F, F)),
        ],
        out_specs=_row_spec(F),
        out_shape=jax.ShapeDtypeStruct((N, F), jnp.float32),
    )(S, g, dinv, b, W_next)


def _head_body(S_ref, g_ref, dinv_ref, b_ref, w1_ref, b1_ref, w2_ref, b2_ref,
               out_ref):
    ssum = S_ref[0] + S_ref[1] + g_ref[...]
    h = jnp.maximum(dinv_ref[...] * ssum + b_ref[...], 0.0)
    z = jnp.maximum(jnp.dot(h, w1_ref[...],
                            preferred_element_type=jnp.float32) + b1_ref[...],
                    0.0)
    y = jnp.dot(z, w2_ref[...], preferred_element_type=jnp.float32)
    out_ref[...] = y[:, 0:1] + b2_ref[0, 0]


def _tc_head(S, g, dinv, b, W_out1, b_out1, W_out2p, b_out2):
    return pl.pallas_call(
        _head_body,
        grid=(_GRID,),
        in_specs=[
            pl.BlockSpec((NC, _RB, F), lambda i: (0, i, 0)),
            _row_spec(F),
            _row_spec(F),
            _full_spec((F,)),
            _full_spec((F, 256)),
            _full_spec((256,)),
            _full_spec((256, F)),
            _full_spec((1, 1)),
        ],
        out_specs=_row_spec(1),
        out_shape=jax.ShapeDtypeStruct((N, 1), jnp.float32),
    )(S, g, dinv, b, W_out1, b_out1, W_out2p, b_out2)


def kernel(x, grid, edge_index, edge_attr,
           W_in, b_in, W1, b1, W2, b2, W3, b3, W4, b4,
           W_out1, b_out1, W_out2, b_out2):
    del edge_attr
    xg = jnp.concatenate([x, grid], axis=-1)
    npad = E_PAD - E
    src = jnp.concatenate(
        [edge_index[0], jnp.zeros((npad,), edge_index.dtype)]
    ).reshape(NW, NBATCH, EB)
    dst = jnp.concatenate(
        [edge_index[1], jnp.full((npad,), N, edge_index.dtype)]
    ).reshape(NW, NBATCH, EB)
    W_out2p = jnp.pad(W_out2, ((0, 0), (0, F - 1)))
    b_out2r = b_out2.reshape(1, 1)

    zrows = jnp.zeros((RPT, F), jnp.float32)
    degp = _sc_scatter(jnp.ones((N, F), jnp.float32), src, dst, zrows)
    g, dinv = _tc_entry(xg, W_in, b_in, W1, degp)

    convs = [(W1, b1), (W2, b2), (W3, b3), (W4, b4)] * 2
    for k in range(8):
        S = _sc_scatter(g, src, dst, zrows)
        _, bk = convs[k]
        if k < 7:
            g = _tc_fused(S, g, dinv, bk, convs[k + 1][0])
        else:
            y = _tc_head(S, g, dinv, bk, W_out1, b_out1, W_out2p, b_out2r)
    return y


# speedup vs baseline: 2.6859x; 2.6859x over previous
import functools

import jax
import jax.numpy as jnp
from jax import lax
from jax.experimental import pallas as pl
from jax.experimental.pallas import tpu as pltpu
from jax.experimental.pallas import tpu_sc as plsc

N = 10000
NP = 10112
E = 320000
F = 128
NC = 2
NS = 16
NW = NC * NS
EB = 128
NBATCH = 80
E_PAD = NW * NBATCH * EB
SB = 8
NSC = NBATCH // SB
RPT = NP // NS

_mesh = plsc.VectorSubcoreMesh(core_axis_name="c", subcore_axis_name="s")


@functools.partial(
    pl.kernel,
    out_type=jax.ShapeDtypeStruct((NC, NP, F), jnp.float32),
    mesh=_mesh,
    scratch_types=[
        pltpu.VMEM((NBATCH, EB), jnp.int32),
        pltpu.VMEM((NBATCH, EB), jnp.int32),
        pltpu.VMEM((EB, F), jnp.float32),
        pltpu.VMEM_SHARED((NP, F), jnp.float32),
        pltpu.SemaphoreType.DMA,
    ],
)
def _sc_scatter(g_hbm, src_hbm, dst_hbm, z_hbm, out_hbm,
                src_i, dst_i, rows_v, acc, gsem):
    c = lax.axis_index("c")
    s = lax.axis_index("s")
    wid = s * NC + c

    pltpu.sync_copy(z_hbm, acc.at[pl.ds(s * RPT, RPT)])
    pltpu.sync_copy(src_hbm.at[wid], src_i)
    pltpu.sync_copy(dst_hbm.at[wid], dst_i)
    plsc.subcore_barrier()

    for i in range(NBATCH):
        pltpu.async_copy(g_hbm.at[src_i.at[i]], rows_v, gsem).wait()
        pltpu.sync_copy(rows_v, acc.at[dst_i.at[i]], add=True)
    plsc.subcore_barrier()

    pltpu.sync_copy(acc.at[pl.ds(s * RPT, RPT)],
                    out_hbm.at[c, pl.ds(s * RPT, RPT)])


_RB = 2000
_GRID = N // _RB


def _row_spec(cols):
    return pl.BlockSpec((_RB, cols), lambda i: (i, 0))


def _full_spec(shape):
    nd = len(shape)
    return pl.BlockSpec(shape, lambda i: (0,) * nd)


def _entry_body(xg_ref, win_ref, bin_ref, wf_ref, degp_ref, g_ref, dinv_ref):
    dp = degp_ref[0, :, 0:1] + degp_ref[1, :, 0:1] + 1.0
    dinv = lax.rsqrt(jnp.maximum(dp, 1.0))
    h = jnp.dot(xg_ref[...], win_ref[...],
                preferred_element_type=jnp.float32) + bin_ref[...]
    g_ref[...] = dinv * jnp.dot(h, wf_ref[...],
                                preferred_element_type=jnp.float32)
    dinv_ref[...] = jnp.broadcast_to(dinv, (_RB, F))


def _tc_entry(xg, W_in, b_in, W_first, degp):
    return pl.pallas_call(
        _entry_body,
        grid=(_GRID,),
        in_specs=[
            _row_spec(12),
            _full_spec((12, F)),
            _full_spec((F,)),
            _full_spec((F, F)),
            pl.BlockSpec((NC, _RB, F), lambda i: (0, i, 0)),
        ],
        out_specs=[_row_spec(F), _row_spec(F)],
        out_shape=[jax.ShapeDtypeStruct((N, F), jnp.float32),
                   jax.ShapeDtypeStruct((N, F), jnp.float32)],
    )(xg, W_in, b_in, W_first, degp)


def _fused_body(S_ref, g_ref, dinv_ref, b_ref, W_ref, out_ref):
    ssum = S_ref[0] + S_ref[1] + g_ref[...]
    h = jnp.maximum(dinv_ref[...] * ssum + b_ref[...], 0.0)
    out_ref[...] = dinv_ref[...] * jnp.dot(h, W_ref[...],
                                           preferred_element_type=jnp.float32)


def _tc_fused(S, g, dinv, b, W_next):
    return pl.pallas_call(
        _fused_body,
        grid=(_GRID,),
        in_specs=[
            pl.BlockSpec((NC, _RB, F), lambda i: (0, i, 0)),
            _row_spec(F),
            _row_spec(F),
            _full_spec((F,)),
            _full_spec((F, F)),
        ],
        out_specs=_row_spec(F),
        out_shape=jax.ShapeDtypeStruct((N, F), jnp.float32),
    )(S, g, dinv, b, W_next)


def _head_body(S_ref, g_ref, dinv_ref, b_ref, w1_ref, b1_ref, w2_ref, b2_ref,
               out_ref):
    ssum = S_ref[0] + S_ref[1] + g_ref[...]
    h = jnp.maximum(dinv_ref[...] * ssum + b_ref[...], 0.0)
    z = jnp.maximum(jnp.dot(h, w1_ref[...],
                            preferred_element_type=jnp.float32) + b1_ref[...],
                    0.0)
    y = jnp.dot(z, w2_ref[...], preferred_element_type=jnp.float32)
    out_ref[...] = y[:, 0:1] + b2_ref[0, 0]


def _tc_head(S, g, dinv, b, W_out1, b_out1, W_out2p, b_out2):
    return pl.pallas_call(
        _head_body,
        grid=(_GRID,),
        in_specs=[
            pl.BlockSpec((NC, _RB, F), lambda i: (0, i, 0)),
            _row_spec(F),
            _row_spec(F),
            _full_spec((F,)),
            _full_spec((F, 256)),
            _full_spec((256,)),
            _full_spec((256, F)),
            _full_spec((1, 1)),
        ],
        out_specs=_row_spec(1),
        out_shape=jax.ShapeDtypeStruct((N, 1), jnp.float32),
    )(S, g, dinv, b, W_out1, b_out1, W_out2p, b_out2)


def kernel(x, grid, edge_index, edge_attr,
           W_in, b_in, W1, b1, W2, b2, W3, b3, W4, b4,
           W_out1, b_out1, W_out2, b_out2):
    del edge_attr
    xg = jnp.concatenate([x, grid], axis=-1)
    npad = E_PAD - E
    pad_ids = jnp.arange(npad, dtype=edge_index.dtype)
    src = jnp.concatenate(
        [edge_index[0], pad_ids % N]
    ).reshape(NW, NBATCH, EB)
    dst = jnp.concatenate(
        [edge_index[1], N + pad_ids % (NP - N)]
    ).reshape(NW, NBATCH, EB)
    W_out2p = jnp.pad(W_out2, ((0, 0), (0, F - 1)))
    b_out2r = b_out2.reshape(1, 1)

    zrows = jnp.zeros((RPT, F), jnp.float32)
    degp = _sc_scatter(jnp.ones((N, F), jnp.float32), src, dst, zrows)
    g, dinv = _tc_entry(xg, W_in, b_in, W1, degp)

    convs = [(W1, b1), (W2, b2), (W3, b3), (W4, b4)] * 2
    for k in range(8):
        S = _sc_scatter(g, src, dst, zrows)
        _, bk = convs[k]
        if k < 7:
            g = _tc_fused(S, g, dinv, bk, convs[k + 1][0])
        else:
            y = _tc_head(S, g, dinv, bk, W_out1, b_out1, W_out2p, b_out2r)
    return y


# speedup vs baseline: 4.1713x; 1.5531x over previous
import functools

import jax
import jax.numpy as jnp
from jax import lax
from jax.experimental import pallas as pl
from jax.experimental.pallas import tpu as pltpu
from jax.experimental.pallas import tpu_sc as plsc

N = 10000
NP = 10112
E = 320000
F = 128
NC = 2
NS = 16
NW = NC * NS
EB = 128
NBATCH = 80
E_PAD = NW * NBATCH * EB
SB = 8
NSC = NBATCH // SB
RPT = NP // NS

_mesh = plsc.VectorSubcoreMesh(core_axis_name="c", subcore_axis_name="s")


@functools.partial(
    pl.kernel,
    out_type=jax.ShapeDtypeStruct((NC, NP, F), jnp.float32),
    mesh=_mesh,
    scratch_types=[
        pltpu.VMEM((NBATCH, EB), jnp.int32),
        pltpu.VMEM((2, SB, EB), jnp.int32),
        pltpu.VMEM((2, EB, F), jnp.float32),
        pltpu.VMEM_SHARED((NP, F), jnp.float32),
        pltpu.SemaphoreType.DMA,
        pltpu.SemaphoreType.DMA,
        pltpu.SemaphoreType.DMA,
        pltpu.SemaphoreType.DMA,
    ],
)
def _sc_scatter(g_hbm, src_hbm, dst_hbm, z_hbm, out_hbm,
                dst_i, src_ib, rows_v, acc, gsem0, gsem1, isem0, isem1):
    c = lax.axis_index("c")
    s = lax.axis_index("s")
    wid = s * NC + c
    gsems = [gsem0, gsem1]
    isems = [isem0, isem1]

    pltpu.sync_copy(z_hbm, acc.at[pl.ds(s * RPT, RPT)])
    pltpu.sync_copy(dst_hbm.at[wid], dst_i)
    pltpu.sync_copy(src_hbm.at[wid, pl.ds(0, SB)], src_ib.at[0])
    pltpu.async_copy(src_hbm.at[wid, pl.ds(SB, SB)], src_ib.at[1], isems[1])
    pltpu.async_copy(g_hbm.at[src_ib.at[0, 0]], rows_v.at[0], gsems[0])
    pltpu.async_copy(g_hbm.at[src_ib.at[0, 1]], rows_v.at[1], gsems[1])
    plsc.subcore_barrier()

    def pair(t, _):
        for half in range(2):
            sc = t * 2 + half
            basei = sc * SB

            @pl.when(sc + 1 < NSC)
            def _():
                pltpu.make_async_copy(src_hbm.at[wid, pl.ds(0, SB)],
                                      src_ib.at[1 - half],
                                      isems[1 - half]).wait()

            for b in range(SB):
                i = basei + b
                slot = b % 2
                pltpu.make_async_copy(g_hbm.at[pl.ds(0, EB)],
                                      rows_v.at[slot], gsems[slot]).wait()
                pltpu.sync_copy(rows_v.at[slot], acc.at[dst_i.at[i]],
                                add=True)
                if b < SB - 2:
                    srcrow = src_ib.at[half, b + 2]
                else:
                    srcrow = src_ib.at[1 - half, b - (SB - 2)]

                @pl.when(i + 2 < NBATCH)
                def _(srcrow=srcrow, slot=slot):
                    pltpu.async_copy(g_hbm.at[srcrow], rows_v.at[slot],
                                     gsems[slot])

            @pl.when(sc + 2 < NSC)
            def _(half=half, sc=sc):
                off = pl.multiple_of((sc + 2) * SB, SB)
                pltpu.async_copy(src_hbm.at[wid, pl.ds(off, SB)],
                                 src_ib.at[half], isems[half])
        return 0
    lax.fori_loop(0, NSC // 2, pair, 0)
    plsc.subcore_barrier()

    pltpu.sync_copy(acc.at[pl.ds(s * RPT, RPT)],
                    out_hbm.at[c, pl.ds(s * RPT, RPT)])


_RB = 2000
_GRID = N // _RB


def _row_spec(cols):
    return pl.BlockSpec((_RB, cols), lambda i: (i, 0))


def _full_spec(shape):
    nd = len(shape)
    return pl.BlockSpec(shape, lambda i: (0,) * nd)


def _entry_body(xg_ref, win_ref, bin_ref, wf_ref, degp_ref, g_ref, dinv_ref):
    dp = degp_ref[0, :, 0:1] + degp_ref[1, :, 0:1] + 1.0
    dinv = lax.rsqrt(jnp.maximum(dp, 1.0))
    h = jnp.dot(xg_ref[...], win_ref[...],
                preferred_element_type=jnp.float32) + bin_ref[...]
    g_ref[...] = dinv * jnp.dot(h, wf_ref[...],
                                preferred_element_type=jnp.float32)
    dinv_ref[...] = jnp.broadcast_to(dinv, (_RB, F))


def _tc_entry(xg, W_in, b_in, W_first, degp):
    return pl.pallas_call(
        _entry_body,
        grid=(_GRID,),
        in_specs=[
            _row_spec(12),
            _full_spec((12, F)),
            _full_spec((F,)),
            _full_spec((F, F)),
            pl.BlockSpec((NC, _RB, F), lambda i: (0, i, 0)),
        ],
        out_specs=[_row_spec(F), _row_spec(F)],
        out_shape=[jax.ShapeDtypeStruct((N, F), jnp.float32),
                   jax.ShapeDtypeStruct((N, F), jnp.float32)],
    )(xg, W_in, b_in, W_first, degp)


def _fused_body(S_ref, g_ref, dinv_ref, b_ref, W_ref, out_ref):
    ssum = S_ref[0] + S_ref[1] + g_ref[...]
    h = jnp.maximum(dinv_ref[...] * ssum + b_ref[...], 0.0)
    out_ref[...] = dinv_ref[...] * jnp.dot(h, W_ref[...],
                                           preferred_element_type=jnp.float32)


def _tc_fused(S, g, dinv, b, W_next):
    return pl.pallas_call(
        _fused_body,
        grid=(_GRID,),
        in_specs=[
            pl.BlockSpec((NC, _RB, F), lambda i: (0, i, 0)),
            _row_spec(F),
            _row_spec(F),
            _full_spec((F,)),
            _full_spec((F, F)),
        ],
        out_specs=_row_spec(F),
        out_shape=jax.ShapeDtypeStruct((N, F), jnp.float32),
    )(S, g, dinv, b, W_next)


def _head_body(S_ref, g_ref, dinv_ref, b_ref, w1_ref, b1_ref, w2_ref, b2_ref,
               out_ref):
    ssum = S_ref[0] + S_ref[1] + g_ref[...]
    h = jnp.maximum(dinv_ref[...] * ssum + b_ref[...], 0.0)
    z = jnp.maximum(jnp.dot(h, w1_ref[...],
                            preferred_element_type=jnp.float32) + b1_ref[...],
                    0.0)
    y = jnp.dot(z, w2_ref[...], preferred_element_type=jnp.float32)
    out_ref[...] = y[:, 0:1] + b2_ref[0, 0]


def _tc_head(S, g, dinv, b, W_out1, b_out1, W_out2p, b_out2):
    return pl.pallas_call(
        _head_body,
        grid=(_GRID,),
        in_specs=[
            pl.BlockSpec((NC, _RB, F), lambda i: (0, i, 0)),
            _row_spec(F),
            _row_spec(F),
            _full_spec((F,)),
            _full_spec((F, 256)),
            _full_spec((256,)),
            _full_spec((256, F)),
            _full_spec((1, 1)),
        ],
        out_specs=_row_spec(1),
        out_shape=jax.ShapeDtypeStruct((N, 1), jnp.float32),
    )(S, g, dinv, b, W_out1, b_out1, W_out2p, b_out2)


def kernel(x, grid, edge_index, edge_attr,
           W_in, b_in, W1, b1, W2, b2, W3, b3, W4, b4,
           W_out1, b_out1, W_out2, b_out2):
    del edge_attr
    xg = jnp.concatenate([x, grid], axis=-1)
    npad = E_PAD - E
    pad_ids = jnp.arange(npad, dtype=edge_index.dtype)
    src = jnp.concatenate(
        [edge_index[0], pad_ids % N]
    ).reshape(NW, NBATCH, EB)
    dst = jnp.concatenate(
        [edge_index[1], N + pad_ids % (NP - N)]
    ).reshape(NW, NBATCH, EB)
    W_out2p = jnp.pad(W_out2, ((0, 0), (0, F - 1)))
    b_out2r = b_out2.reshape(1, 1)

    zrows = jnp.zeros((RPT, F), jnp.float32)
    degp = _sc_scatter(jnp.ones((N, F), jnp.float32), src, dst, zrows)
    g, dinv = _tc_entry(xg, W_in, b_in, W1, degp)

    convs = [(W1, b1), (W2, b2), (W3, b3), (W4, b4)] * 2
    for k in range(8):
        S = _sc_scatter(g, src, dst, zrows)
        _, bk = convs[k]
        if k < 7:
            g = _tc_fused(S, g, dinv, bk, convs[k + 1][0])
        else:
            y = _tc_head(S, g, dinv, bk, W_out1, b_out1, W_out2p, b_out2r)
    return y
